# baseline (speedup 1.0000x reference)
import functools

import jax
import jax.numpy as jnp
from jax import lax
from jax.experimental import pallas as pl
from jax.experimental.pallas import tpu as pltpu
from jax.experimental.pallas import tpu_sc as plsc

_K = 8192
_D = 32
_COMMIT = 0.25

_R = 256
_KC = 2048


def _dist_argmin_kernel(z_ref, cb_ref, idx_ref, dsum_ref):
    i = pl.program_id(0)
    zb = z_ref[...]
    a = jnp.sum(zb * zb, axis=1, keepdims=True)

    def body(j, carry):
        m_best, i_best = carry
        cb = cb_ref[pl.ds(j * _KC, _KC), :]
        bb = jnp.sum(cb * cb, axis=1)
        mm = lax.dot_general(zb, cb, (((1,), (1,)), ((), ())))
        d = (a + bb[None, :]) - 2.0 * mm
        cm = jnp.min(d, axis=1, keepdims=True)
        iota = lax.broadcasted_iota(jnp.int32, d.shape, 1) + j * _KC
        ci = jnp.min(jnp.where(d == cm, iota, _K), axis=1, keepdims=True)
        upd = cm < m_best
        return (jnp.where(upd, cm, m_best), jnp.where(upd, ci, i_best))

    init = (jnp.full((_R, 1), jnp.inf, jnp.float32),
            jnp.zeros((_R, 1), jnp.int32))
    m_best, i_best = lax.fori_loop(0, _K // _KC, body, init)

    idx_ref[...] = i_best

    @pl.when(i == 0)
    def _():
        dsum_ref[...] = jnp.zeros((1, 1), jnp.float32)

    dsum_ref[...] += jnp.sum(m_best).reshape(1, 1)


def _dist_argmin(z_flat, codebook):
    n = z_flat.shape[0]
    grid = n // _R
    idx, dsum = pl.pallas_call(
        _dist_argmin_kernel,
        grid=(grid,),
        in_specs=[
            pl.BlockSpec((_R, _D), lambda i: (i, 0)),
            pl.BlockSpec((_K, _D), lambda i: (0, 0)),
        ],
        out_specs=[
            pl.BlockSpec((_R, 1), lambda i: (i, 0)),
            pl.BlockSpec((1, 1), lambda i: (0, 0)),
        ],
        out_shape=[
            jax.ShapeDtypeStruct((n, 1), jnp.int32),
            jax.ShapeDtypeStruct((1, 1), jnp.float32),
        ],
        compiler_params=pltpu.CompilerParams(
            dimension_semantics=("arbitrary",),
        ),
    )(z_flat, codebook)
    return idx.reshape(n), dsum[0, 0]


def _make_sc_gather(n):
    info = plsc.get_sparse_core_info()
    nw = info.num_cores * info.num_subcores
    b_per_w = n // nw
    mesh = plsc.VectorSubcoreMesh(core_axis_name="c", subcore_axis_name="s")

    @functools.partial(
        pl.kernel, mesh=mesh,
        compiler_params=pltpu.CompilerParams(use_tc_tiling_on_sc=False),
        out_type=jax.ShapeDtypeStruct((n, _D), jnp.float32),
        scratch_types=[
            pltpu.VMEM((b_per_w,), jnp.int32),
            pltpu.VMEM((b_per_w, _D), jnp.float32),
            pltpu.SemaphoreType.DMA,
        ],
    )
    def gather(cb_hbm, idx_hbm, out_hbm, idx_v, rows_v, sem):
        wid = lax.axis_index("s") * info.num_cores + lax.axis_index("c")
        base = wid * b_per_w
        pltpu.sync_copy(idx_hbm.at[pl.ds(base, b_per_w)], idx_v)
        pltpu.async_copy(cb_hbm.at[idx_v], rows_v, sem).wait()
        pltpu.sync_copy(rows_v, out_hbm.at[pl.ds(base, b_per_w)])

    return gather


def kernel(z, codebook):
    z = z.astype(jnp.float32)
    zt = jnp.transpose(z, (0, 2, 3, 1))
    b, h, w, c = zt.shape
    z_flat = zt.reshape(-1, c)

    idx, dsum = _dist_argmin(z_flat, codebook)

    zq_flat = _make_sc_gather(z_flat.shape[0])(codebook, idx)

    mse = dsum / jnp.float32(z_flat.size)
    loss = _COMMIT * mse + mse

    zq = z_flat + lax.stop_gradient(zq_flat - z_flat)
    zq = jnp.transpose(zq.reshape(b, h, w, c), (0, 3, 1, 2))
    return (zq, loss, idx.reshape(b, h, w))


# speedup vs baseline: 1.0161x; 1.0161x over previous
import functools

import jax
import jax.numpy as jnp
from jax import lax
from jax.experimental import pallas as pl
from jax.experimental.pallas import tpu as pltpu
from jax.experimental.pallas import tpu_sc as plsc

_K = 8192
_D = 32
_COMMIT = 0.25

_R = 256
_KC = 2048


def _dist_argmin_kernel(z_ref, cb_ref, idx_ref, dsum_ref, bb_ref):
    i = pl.program_id(0)
    zb = z_ref[...]
    a = jnp.sum(zb * zb, axis=1, keepdims=True)

    @pl.when(i == 0)
    def _():
        cb = cb_ref[...]
        bb_ref[...] = jnp.sum(cb * cb, axis=1).reshape(1, _K)

    def body(j, carry):
        m_best, i_best = carry
        cb = cb_ref[pl.ds(j * _KC, _KC), :]
        bb = bb_ref[0, pl.ds(j * _KC, _KC)]
        mm = lax.dot_general(zb, cb, (((1,), (1,)), ((), ())))
        d = (a + bb[None, :]) - 2.0 * mm
        cm = jnp.min(d, axis=1, keepdims=True)
        iota = lax.broadcasted_iota(jnp.int32, d.shape, 1) + j * _KC
        ci = jnp.min(jnp.where(d == cm, iota, _K), axis=1, keepdims=True)
        upd = cm < m_best
        return (jnp.where(upd, cm, m_best), jnp.where(upd, ci, i_best))

    init = (jnp.full((_R, 1), jnp.inf, jnp.float32),
            jnp.zeros((_R, 1), jnp.int32))
    m_best, i_best = lax.fori_loop(0, _K // _KC, body, init)

    idx_ref[...] = i_best

    @pl.when(i == 0)
    def _():
        dsum_ref[...] = jnp.zeros((1, 1), jnp.float32)

    dsum_ref[...] += jnp.sum(m_best).reshape(1, 1)


def _dist_argmin(z_flat, codebook):
    n = z_flat.shape[0]
    grid = n // _R
    idx, dsum = pl.pallas_call(
        _dist_argmin_kernel,
        grid=(grid,),
        in_specs=[
            pl.BlockSpec((_R, _D), lambda i: (i, 0)),
            pl.BlockSpec((_K, _D), lambda i: (0, 0)),
        ],
        out_specs=[
            pl.BlockSpec((_R, 1), lambda i: (i, 0)),
            pl.BlockSpec((1, 1), lambda i: (0, 0)),
        ],
        out_shape=[
            jax.ShapeDtypeStruct((n, 1), jnp.int32),
            jax.ShapeDtypeStruct((1, 1), jnp.float32),
        ],
        scratch_shapes=[pltpu.VMEM((1, _K), jnp.float32)],
        compiler_params=pltpu.CompilerParams(
            dimension_semantics=("arbitrary",),
            vmem_limit_bytes=100 * 1024 * 1024,
        ),
    )(z_flat, codebook)
    return idx.reshape(n), dsum[0, 0]


def _make_sc_gather(n):
    info = plsc.get_sparse_core_info()
    nw = info.num_cores * info.num_subcores
    b_per_w = n // nw
    mesh = plsc.VectorSubcoreMesh(core_axis_name="c", subcore_axis_name="s")

    @functools.partial(
        pl.kernel, mesh=mesh,
        compiler_params=pltpu.CompilerParams(use_tc_tiling_on_sc=False),
        out_type=jax.ShapeDtypeStruct((n, _D), jnp.float32),
        scratch_types=[
            pltpu.VMEM((b_per_w,), jnp.int32),
            pltpu.VMEM((b_per_w, _D), jnp.float32),
            pltpu.SemaphoreType.DMA,
        ],
    )
    def gather(cb_hbm, idx_hbm, out_hbm, idx_v, rows_v, sem):
        wid = lax.axis_index("s") * info.num_cores + lax.axis_index("c")
        base = wid * b_per_w
        pltpu.sync_copy(idx_hbm.at[pl.ds(base, b_per_w)], idx_v)
        pltpu.async_copy(cb_hbm.at[idx_v], rows_v, sem).wait()
        pltpu.sync_copy(rows_v, out_hbm.at[pl.ds(base, b_per_w)])

    return gather


def kernel(z, codebook):
    z = z.astype(jnp.float32)
    zt = jnp.transpose(z, (0, 2, 3, 1))
    b, h, w, c = zt.shape
    z_flat = zt.reshape(-1, c)

    idx, dsum = _dist_argmin(z_flat, codebook)

    zq_flat = _make_sc_gather(z_flat.shape[0])(codebook, idx)

    mse = dsum / jnp.float32(z_flat.size)
    loss = _COMMIT * mse + mse

    zq = z_flat + lax.stop_gradient(zq_flat - z_flat)
    zq = jnp.transpose(zq.reshape(b, h, w, c), (0, 3, 1, 2))
    return (zq, loss, idx.reshape(b, h, w))


# speedup vs baseline: 1.1577x; 1.1394x over previous
import functools

import jax
import jax.numpy as jnp
from jax import lax
from jax.experimental import pallas as pl
from jax.experimental.pallas import tpu as pltpu
from jax.experimental.pallas import tpu_sc as plsc

_K = 8192
_D = 32
_COMMIT = 0.25

_R = 256
_KC = 4096


def _dist_argmin_kernel(z_ref, cb_ref, idx_ref, dsum_ref, bb_ref):
    i = pl.program_id(0)
    zb = z_ref[...]
    a = jnp.sum(zb * zb, axis=1, keepdims=True)

    @pl.when(i == 0)
    def _():
        cb = cb_ref[...]
        bb_ref[...] = jnp.sum(cb * cb, axis=1).reshape(1, _K)

    def body(j, carry):
        m_best, i_best = carry
        cb = cb_ref[pl.ds(j * _KC, _KC), :]
        bb = bb_ref[0, pl.ds(j * _KC, _KC)]
        mm = lax.dot_general(zb, cb, (((1,), (1,)), ((), ())))
        d = (a + bb[None, :]) - 2.0 * mm
        cm = jnp.min(d, axis=1, keepdims=True)
        iota = lax.broadcasted_iota(jnp.int32, d.shape, 1) + j * _KC
        ci = jnp.min(jnp.where(d == cm, iota, _K), axis=1, keepdims=True)
        upd = cm < m_best
        return (jnp.where(upd, cm, m_best), jnp.where(upd, ci, i_best))

    init = (jnp.full((_R, 1), jnp.inf, jnp.float32),
            jnp.zeros((_R, 1), jnp.int32))
    m_best, i_best = lax.fori_loop(0, _K // _KC, body, init)

    idx_ref[...] = i_best

    @pl.when(i == 0)
    def _():
        dsum_ref[...] = jnp.zeros((1, 1), jnp.float32)

    dsum_ref[...] += jnp.sum(m_best).reshape(1, 1)


def _dist_argmin(z_flat, codebook):
    n = z_flat.shape[0]
    grid = n // _R
    idx, dsum = pl.pallas_call(
        _dist_argmin_kernel,
        grid=(grid,),
        in_specs=[
            pl.BlockSpec((_R, _D), lambda i: (i, 0)),
            pl.BlockSpec((_K, _D), lambda i: (0, 0)),
        ],
        out_specs=[
            pl.BlockSpec((_R, 1), lambda i: (i, 0)),
            pl.BlockSpec((1, 1), lambda i: (0, 0)),
        ],
        out_shape=[
            jax.ShapeDtypeStruct((n, 1), jnp.int32),
            jax.ShapeDtypeStruct((1, 1), jnp.float32),
        ],
        scratch_shapes=[pltpu.VMEM((1, _K), jnp.float32)],
        compiler_params=pltpu.CompilerParams(
            dimension_semantics=("arbitrary",),
            vmem_limit_bytes=100 * 1024 * 1024,
        ),
    )(z_flat, codebook)
    return idx.reshape(n), dsum[0, 0]


def _make_sc_gather(n):
    info = plsc.get_sparse_core_info()
    nw = info.num_cores * info.num_subcores
    b_per_w = n // nw
    mesh = plsc.VectorSubcoreMesh(core_axis_name="c", subcore_axis_name="s")

    @functools.partial(
        pl.kernel, mesh=mesh,
        compiler_params=pltpu.CompilerParams(use_tc_tiling_on_sc=False),
        out_type=jax.ShapeDtypeStruct((n, _D), jnp.float32),
        scratch_types=[
            pltpu.VMEM((b_per_w,), jnp.int32),
            pltpu.VMEM((b_per_w, _D), jnp.float32),
            pltpu.SemaphoreType.DMA,
        ],
    )
    def gather(cb_hbm, idx_hbm, out_hbm, idx_v, rows_v, sem):
        wid = lax.axis_index("s") * info.num_cores + lax.axis_index("c")
        base = wid * b_per_w
        pltpu.sync_copy(idx_hbm.at[pl.ds(base, b_per_w)], idx_v)
        pltpu.async_copy(cb_hbm.at[idx_v], rows_v, sem).wait()
        pltpu.sync_copy(rows_v, out_hbm.at[pl.ds(base, b_per_w)])

    return gather


def kernel(z, codebook):
    z = z.astype(jnp.float32)
    zt = jnp.transpose(z, (0, 2, 3, 1))
    b, h, w, c = zt.shape
    z_flat = zt.reshape(-1, c)

    idx, dsum = _dist_argmin(z_flat, codebook)

    zq_flat = _make_sc_gather(z_flat.shape[0])(codebook, idx)

    mse = dsum / jnp.float32(z_flat.size)
    loss = _COMMIT * mse + mse

    zq = z_flat + lax.stop_gradient(zq_flat - z_flat)
    zq = jnp.transpose(zq.reshape(b, h, w, c), (0, 3, 1, 2))
    return (zq, loss, idx.reshape(b, h, w))


# speedup vs baseline: 1.2784x; 1.1042x over previous
import functools

import jax
import jax.numpy as jnp
from jax import lax
from jax.experimental import pallas as pl
from jax.experimental.pallas import tpu as pltpu
from jax.experimental.pallas import tpu_sc as plsc

_K = 8192
_D = 32
_COMMIT = 0.25

_R = 256
_KC = 8192


def _dist_argmin_kernel(z_ref, cb_ref, idx_ref, dsum_ref, bb_ref):
    i = pl.program_id(0)
    zb = z_ref[...]
    a = jnp.sum(zb * zb, axis=1, keepdims=True)

    @pl.when(i == 0)
    def _():
        cb = cb_ref[...]
        bb_ref[...] = jnp.sum(cb * cb, axis=1).reshape(1, _K)

    def body(j, carry):
        m_best, i_best = carry
        cb = cb_ref[pl.ds(j * _KC, _KC), :]
        bb = bb_ref[0, pl.ds(j * _KC, _KC)]
        mm = lax.dot_general(zb, cb, (((1,), (1,)), ((), ())))
        d = (a + bb[None, :]) - 2.0 * mm
        cm = jnp.min(d, axis=1, keepdims=True)
        iota = lax.broadcasted_iota(jnp.int32, d.shape, 1) + j * _KC
        ci = jnp.min(jnp.where(d == cm, iota, _K), axis=1, keepdims=True)
        upd = cm < m_best
        return (jnp.where(upd, cm, m_best), jnp.where(upd, ci, i_best))

    init = (jnp.full((_R, 1), jnp.inf, jnp.float32),
            jnp.zeros((_R, 1), jnp.int32))
    m_best, i_best = lax.fori_loop(0, _K // _KC, body, init)

    idx_ref[...] = i_best

    @pl.when(i == 0)
    def _():
        dsum_ref[...] = jnp.zeros((1, 1), jnp.float32)

    dsum_ref[...] += jnp.sum(m_best).reshape(1, 1)


def _dist_argmin(z_flat, codebook):
    n = z_flat.shape[0]
    grid = n // _R
    idx, dsum = pl.pallas_call(
        _dist_argmin_kernel,
        grid=(grid,),
        in_specs=[
            pl.BlockSpec((_R, _D), lambda i: (i, 0)),
            pl.BlockSpec((_K, _D), lambda i: (0, 0)),
        ],
        out_specs=[
            pl.BlockSpec((_R, 1), lambda i: (i, 0)),
            pl.BlockSpec((1, 1), lambda i: (0, 0)),
        ],
        out_shape=[
            jax.ShapeDtypeStruct((n, 1), jnp.int32),
            jax.ShapeDtypeStruct((1, 1), jnp.float32),
        ],
        scratch_shapes=[pltpu.VMEM((1, _K), jnp.float32)],
        compiler_params=pltpu.CompilerParams(
            dimension_semantics=("arbitrary",),
            vmem_limit_bytes=100 * 1024 * 1024,
        ),
    )(z_flat, codebook)
    return idx.reshape(n), dsum[0, 0]


def _make_sc_gather(n):
    info = plsc.get_sparse_core_info()
    nw = info.num_cores * info.num_subcores
    b_per_w = n // nw
    mesh = plsc.VectorSubcoreMesh(core_axis_name="c", subcore_axis_name="s")

    @functools.partial(
        pl.kernel, mesh=mesh,
        compiler_params=pltpu.CompilerParams(use_tc_tiling_on_sc=False),
        out_type=jax.ShapeDtypeStruct((n, _D), jnp.float32),
        scratch_types=[
            pltpu.VMEM((b_per_w,), jnp.int32),
            pltpu.VMEM((b_per_w, _D), jnp.float32),
            pltpu.SemaphoreType.DMA,
        ],
    )
    def gather(cb_hbm, idx_hbm, out_hbm, idx_v, rows_v, sem):
        wid = lax.axis_index("s") * info.num_cores + lax.axis_index("c")
        base = wid * b_per_w
        pltpu.sync_copy(idx_hbm.at[pl.ds(base, b_per_w)], idx_v)
        pltpu.async_copy(cb_hbm.at[idx_v], rows_v, sem).wait()
        pltpu.sync_copy(rows_v, out_hbm.at[pl.ds(base, b_per_w)])

    return gather


def kernel(z, codebook):
    z = z.astype(jnp.float32)
    zt = jnp.transpose(z, (0, 2, 3, 1))
    b, h, w, c = zt.shape
    z_flat = zt.reshape(-1, c)

    idx, dsum = _dist_argmin(z_flat, codebook)

    zq_flat = _make_sc_gather(z_flat.shape[0])(codebook, idx)

    mse = dsum / jnp.float32(z_flat.size)
    loss = _COMMIT * mse + mse

    zq = z_flat + lax.stop_gradient(zq_flat - z_flat)
    zq = jnp.transpose(zq.reshape(b, h, w, c), (0, 3, 1, 2))
    return (zq, loss, idx.reshape(b, h, w))


# speedup vs baseline: 1.3812x; 1.0805x over previous
import functools

import jax
import jax.numpy as jnp
from jax import lax
from jax.experimental import pallas as pl
from jax.experimental.pallas import tpu as pltpu
from jax.experimental.pallas import tpu_sc as plsc

_K = 8192
_D = 32
_COMMIT = 0.25

_R = 512
_KC = 8192


def _dist_argmin_kernel(z_ref, cb_ref, idx_ref, dsum_ref, bb_ref):
    i = pl.program_id(0)
    zb = z_ref[...]
    a = jnp.sum(zb * zb, axis=1, keepdims=True)

    @pl.when(i == 0)
    def _():
        cb = cb_ref[...]
        bb_ref[...] = jnp.sum(cb * cb, axis=1).reshape(1, _K)

    def body(j, carry):
        m_best, i_best = carry
        cb = cb_ref[pl.ds(j * _KC, _KC), :]
        bb = bb_ref[0, pl.ds(j * _KC, _KC)]
        mm = lax.dot_general(zb, cb, (((1,), (1,)), ((), ())))
        d = (a + bb[None, :]) - 2.0 * mm
        cm = jnp.min(d, axis=1, keepdims=True)
        iota = lax.broadcasted_iota(jnp.int32, d.shape, 1) + j * _KC
        ci = jnp.min(jnp.where(d == cm, iota, _K), axis=1, keepdims=True)
        upd = cm < m_best
        return (jnp.where(upd, cm, m_best), jnp.where(upd, ci, i_best))

    init = (jnp.full((_R, 1), jnp.inf, jnp.float32),
            jnp.zeros((_R, 1), jnp.int32))
    m_best, i_best = lax.fori_loop(0, _K // _KC, body, init)

    idx_ref[...] = i_best

    @pl.when(i == 0)
    def _():
        dsum_ref[...] = jnp.zeros((1, 1), jnp.float32)

    dsum_ref[...] += jnp.sum(m_best).reshape(1, 1)


def _dist_argmin(z_flat, codebook):
    n = z_flat.shape[0]
    grid = n // _R
    idx, dsum = pl.pallas_call(
        _dist_argmin_kernel,
        grid=(grid,),
        in_specs=[
            pl.BlockSpec((_R, _D), lambda i: (i, 0)),
            pl.BlockSpec((_K, _D), lambda i: (0, 0)),
        ],
        out_specs=[
            pl.BlockSpec((_R, 1), lambda i: (i, 0)),
            pl.BlockSpec((1, 1), lambda i: (0, 0)),
        ],
        out_shape=[
            jax.ShapeDtypeStruct((n, 1), jnp.int32),
            jax.ShapeDtypeStruct((1, 1), jnp.float32),
        ],
        scratch_shapes=[pltpu.VMEM((1, _K), jnp.float32)],
        compiler_params=pltpu.CompilerParams(
            dimension_semantics=("arbitrary",),
            vmem_limit_bytes=100 * 1024 * 1024,
        ),
    )(z_flat, codebook)
    return idx.reshape(n), dsum[0, 0]


def _make_sc_gather(n):
    info = plsc.get_sparse_core_info()
    nw = info.num_cores * info.num_subcores
    b_per_w = n // nw
    mesh = plsc.VectorSubcoreMesh(core_axis_name="c", subcore_axis_name="s")

    @functools.partial(
        pl.kernel, mesh=mesh,
        compiler_params=pltpu.CompilerParams(use_tc_tiling_on_sc=False),
        out_type=jax.ShapeDtypeStruct((n, _D), jnp.float32),
        scratch_types=[
            pltpu.VMEM((b_per_w,), jnp.int32),
            pltpu.VMEM((b_per_w, _D), jnp.float32),
            pltpu.SemaphoreType.DMA,
        ],
    )
    def gather(cb_hbm, idx_hbm, out_hbm, idx_v, rows_v, sem):
        wid = lax.axis_index("s") * info.num_cores + lax.axis_index("c")
        base = wid * b_per_w
        pltpu.sync_copy(idx_hbm.at[pl.ds(base, b_per_w)], idx_v)
        pltpu.async_copy(cb_hbm.at[idx_v], rows_v, sem).wait()
        pltpu.sync_copy(rows_v, out_hbm.at[pl.ds(base, b_per_w)])

    return gather


def kernel(z, codebook):
    z = z.astype(jnp.float32)
    zt = jnp.transpose(z, (0, 2, 3, 1))
    b, h, w, c = zt.shape
    z_flat = zt.reshape(-1, c)

    idx, dsum = _dist_argmin(z_flat, codebook)

    zq_flat = _make_sc_gather(z_flat.shape[0])(codebook, idx)

    mse = dsum / jnp.float32(z_flat.size)
    loss = _COMMIT * mse + mse

    zq = z_flat + lax.stop_gradient(zq_flat - z_flat)
    zq = jnp.transpose(zq.reshape(b, h, w, c), (0, 3, 1, 2))
    return (zq, loss, idx.reshape(b, h, w))


# speedup vs baseline: 1.3954x; 1.0103x over previous
import functools

import jax
import jax.numpy as jnp
from jax import lax
from jax.experimental import pallas as pl
from jax.experimental.pallas import tpu as pltpu
from jax.experimental.pallas import tpu_sc as plsc

_K = 8192
_D = 32
_COMMIT = 0.25

_R = 1024
_KC = 8192


def _dist_argmin_kernel(z_ref, cb_ref, idx_ref, dsum_ref, bb_ref):
    i = pl.program_id(0)
    zb = z_ref[...]
    a = jnp.sum(zb * zb, axis=1, keepdims=True)

    @pl.when(i == 0)
    def _():
        cb = cb_ref[...]
        bb_ref[...] = jnp.sum(cb * cb, axis=1).reshape(1, _K)

    def body(j, carry):
        m_best, i_best = carry
        cb = cb_ref[pl.ds(j * _KC, _KC), :]
        bb = bb_ref[0, pl.ds(j * _KC, _KC)]
        mm = lax.dot_general(zb, cb, (((1,), (1,)), ((), ())))
        d = (a + bb[None, :]) - 2.0 * mm
        cm = jnp.min(d, axis=1, keepdims=True)
        iota = lax.broadcasted_iota(jnp.int32, d.shape, 1) + j * _KC
        ci = jnp.min(jnp.where(d == cm, iota, _K), axis=1, keepdims=True)
        upd = cm < m_best
        return (jnp.where(upd, cm, m_best), jnp.where(upd, ci, i_best))

    init = (jnp.full((_R, 1), jnp.inf, jnp.float32),
            jnp.zeros((_R, 1), jnp.int32))
    m_best, i_best = lax.fori_loop(0, _K // _KC, body, init)

    idx_ref[...] = i_best

    @pl.when(i == 0)
    def _():
        dsum_ref[...] = jnp.zeros((1, 1), jnp.float32)

    dsum_ref[...] += jnp.sum(m_best).reshape(1, 1)


def _dist_argmin(z_flat, codebook):
    n = z_flat.shape[0]
    grid = n // _R
    idx, dsum = pl.pallas_call(
        _dist_argmin_kernel,
        grid=(grid,),
        in_specs=[
            pl.BlockSpec((_R, _D), lambda i: (i, 0)),
            pl.BlockSpec((_K, _D), lambda i: (0, 0)),
        ],
        out_specs=[
            pl.BlockSpec((_R, 1), lambda i: (i, 0)),
            pl.BlockSpec((1, 1), lambda i: (0, 0)),
        ],
        out_shape=[
            jax.ShapeDtypeStruct((n, 1), jnp.int32),
            jax.ShapeDtypeStruct((1, 1), jnp.float32),
        ],
        scratch_shapes=[pltpu.VMEM((1, _K), jnp.float32)],
        compiler_params=pltpu.CompilerParams(
            dimension_semantics=("arbitrary",),
            vmem_limit_bytes=100 * 1024 * 1024,
        ),
    )(z_flat, codebook)
    return idx.reshape(n), dsum[0, 0]


def _make_sc_gather(n):
    info = plsc.get_sparse_core_info()
    nw = info.num_cores * info.num_subcores
    b_per_w = n // nw
    mesh = plsc.VectorSubcoreMesh(core_axis_name="c", subcore_axis_name="s")

    @functools.partial(
        pl.kernel, mesh=mesh,
        compiler_params=pltpu.CompilerParams(use_tc_tiling_on_sc=False),
        out_type=jax.ShapeDtypeStruct((n, _D), jnp.float32),
        scratch_types=[
            pltpu.VMEM((b_per_w,), jnp.int32),
            pltpu.VMEM((b_per_w, _D), jnp.float32),
            pltpu.SemaphoreType.DMA,
        ],
    )
    def gather(cb_hbm, idx_hbm, out_hbm, idx_v, rows_v, sem):
        wid = lax.axis_index("s") * info.num_cores + lax.axis_index("c")
        base = wid * b_per_w
        pltpu.sync_copy(idx_hbm.at[pl.ds(base, b_per_w)], idx_v)
        pltpu.async_copy(cb_hbm.at[idx_v], rows_v, sem).wait()
        pltpu.sync_copy(rows_v, out_hbm.at[pl.ds(base, b_per_w)])

    return gather


def kernel(z, codebook):
    z = z.astype(jnp.float32)
    zt = jnp.transpose(z, (0, 2, 3, 1))
    b, h, w, c = zt.shape
    z_flat = zt.reshape(-1, c)

    idx, dsum = _dist_argmin(z_flat, codebook)

    zq_flat = _make_sc_gather(z_flat.shape[0])(codebook, idx)

    mse = dsum / jnp.float32(z_flat.size)
    loss = _COMMIT * mse + mse

    zq = z_flat + lax.stop_gradient(zq_flat - z_flat)
    zq = jnp.transpose(zq.reshape(b, h, w, c), (0, 3, 1, 2))
    return (zq, loss, idx.reshape(b, h, w))
